# Initial kernel scaffold; baseline (speedup 1.0000x reference)
import jax
import jax.numpy as jnp
from jax.experimental import pallas as pl


def kernel(x_user, x_movie, edge_index_rates, edge_index_rev, W1l, b1l, W1r, W2l, b2l, W2r, W3l, b3l, W3r, Wlin1, blin1, Wlin2, blin2):
    raise NotImplementedError("write your pallas kernel here")


# speedup vs baseline: 4.0957x; 4.0957x over previous
import functools

import jax
import jax.numpy as jnp
from jax import lax
from jax.experimental import pallas as pl
from jax.experimental.pallas import tpu as pltpu
from jax.experimental.pallas import tpu_sc as plsc

NC = 2
NS = 16
NW = NC * NS
E = 320000
EPW = E // NW
K = 80
CHUNKS = EPW // K
D = 128
NPAD = 10240
RPS = NPAD // NS

_mesh = plsc.VectorSubcoreMesh(core_axis_name="c", subcore_axis_name="s")


def _segsum_body(x, src, dst, zfeat, out_sum, sidx, didx, rows, acc, sem):
  c = lax.axis_index("c")
  s = lax.axis_index("s")
  wid = c * NS + s

  pltpu.sync_copy(zfeat, acc.at[pl.ds(s * RPS, RPS)])
  plsc.subcore_barrier()

  ebase = wid * EPW

  def step(i, carry):
    b = pl.multiple_of(ebase + i * K, 8)
    pltpu.sync_copy(src.at[pl.ds(b, K)], sidx)
    pltpu.sync_copy(dst.at[pl.ds(b, K)], didx)
    pltpu.async_copy(x.at[sidx], rows, sem).wait()
    pltpu.sync_copy(rows, acc.at[didx], add=True)
    return carry

  lax.fori_loop(0, CHUNKS, step, 0)
  plsc.subcore_barrier()

  pltpu.sync_copy(acc.at[pl.ds(s * RPS, RPS)],
                  out_sum.at[c, pl.ds(s * RPS, RPS)])


def _count_body(dst, zfeat, ones, out_cnt, didx, onesv, acc, sem):
  c = lax.axis_index("c")
  s = lax.axis_index("s")
  wid = c * NS + s

  pltpu.sync_copy(zfeat, acc.at[pl.ds(s * RPS, RPS)])
  pltpu.sync_copy(ones, onesv)
  plsc.subcore_barrier()

  ebase = wid * EPW

  def step(i, carry):
    b = pl.multiple_of(ebase + i * K, 8)
    pltpu.sync_copy(dst.at[pl.ds(b, K)], didx)
    pltpu.sync_copy(onesv, acc.at[didx], add=True)
    return carry

  lax.fori_loop(0, CHUNKS, step, 0)
  plsc.subcore_barrier()

  pltpu.sync_copy(acc.at[pl.ds(s * RPS, RPS)],
                  out_cnt.at[c, pl.ds(s * RPS, RPS)])


_segsum_call = functools.partial(
    pl.kernel, mesh=_mesh,
    out_type=[jax.ShapeDtypeStruct((NC, NPAD, D), jnp.float32)],
    scratch_types=[
        pltpu.VMEM((K,), jnp.int32),
        pltpu.VMEM((K,), jnp.int32),
        pltpu.VMEM((K, D), jnp.float32),
        pltpu.VMEM_SHARED((NPAD, D), jnp.float32),
        pltpu.SemaphoreType.DMA,
    ])(_segsum_body)

_count_call = functools.partial(
    pl.kernel, mesh=_mesh,
    out_type=[jax.ShapeDtypeStruct((NC, NPAD, D), jnp.float32)],
    scratch_types=[
        pltpu.VMEM((K,), jnp.int32),
        pltpu.VMEM((K, D), jnp.float32),
        pltpu.VMEM_SHARED((NPAD, D), jnp.float32),
        pltpu.SemaphoreType.DMA,
    ])(_count_body)


def _segsum(x, src, dst):
  zfeat = jnp.zeros((RPS, D), jnp.float32)
  (s,) = _segsum_call(x, src, dst, zfeat)
  return s


def _count(dst):
  zfeat = jnp.zeros((RPS, D), jnp.float32)
  ones = jnp.ones((K, D), jnp.float32)
  (cnt,) = _count_call(dst, zfeat, ones)
  return cnt


R = 1000


def _make_dense(proj):
  def body(*refs):
    if proj:
      (sums, cnts, xd, wl, bl, wr, wp, bp, h_out, p_out) = refs
    else:
      (sums, cnts, xd, wl, bl, wr, h_out) = refs
    ssum = sums[0] + sums[1]
    cnt = cnts[0][:, 0:1] + cnts[1][:, 0:1]
    mean = ssum / jnp.maximum(cnt, 1.0)
    h = lax.dot_general(mean, wl[...], (((1,), (1,)), ((), ())),
                        preferred_element_type=jnp.float32)
    h = h + bl[...]
    h = h + lax.dot_general(xd[...], wr[...], (((1,), (1,)), ((), ())),
                            preferred_element_type=jnp.float32)
    h = jnp.maximum(h, 0.0)
    h_out[...] = h
    if proj:
      p = lax.dot_general(h, wp[...], (((1,), (1,)), ((), ())),
                          preferred_element_type=jnp.float32)
      p_out[...] = p + bp[...]

  n_dst = 10000
  grid = n_dst // R
  in_specs = [
      pl.BlockSpec((NC, R, D), lambda i: (0, i, 0)),
      pl.BlockSpec((NC, R, D), lambda i: (0, i, 0)),
      pl.BlockSpec((R, D), lambda i: (i, 0)),
      pl.BlockSpec((D, D), lambda i: (0, 0)),
      pl.BlockSpec((1, D), lambda i: (0, 0)),
      pl.BlockSpec((D, D), lambda i: (0, 0)),
  ]
  out_shape = [jax.ShapeDtypeStruct((n_dst, D), jnp.float32)]
  out_specs = [pl.BlockSpec((R, D), lambda i: (i, 0))]
  if proj:
    in_specs += [
        pl.BlockSpec((64, D), lambda i: (0, 0)),
        pl.BlockSpec((1, 64), lambda i: (0, 0)),
    ]
    out_shape.append(jax.ShapeDtypeStruct((n_dst, 64), jnp.float32))
    out_specs.append(pl.BlockSpec((R, 64), lambda i: (i, 0)))

  return pl.pallas_call(
      body, grid=(grid,), in_specs=in_specs, out_specs=out_specs,
      out_shape=out_shape)


_dense_plain = _make_dense(False)
_dense_proj = _make_dense(True)


def _dense(sums, cnts, x_dst, wl, bl, wr, wp=None, bp=None):
  bl = bl.reshape(1, D)
  if wp is None:
    (h,) = _dense_plain(sums, cnts, x_dst, wl, bl, wr)
    return h
  bp = bp.reshape(1, 64)
  h, p = _dense_proj(sums, cnts, x_dst, wl, bl, wr, wp, bp)
  return h, p


def kernel(x_user, x_movie, edge_index_rates, edge_index_rev,
           W1l, b1l, W1r, W2l, b2l, W2r, W3l, b3l, W3r,
           Wlin1, blin1, Wlin2, blin2):
  src_rev = edge_index_rev[0]
  dst_rev = edge_index_rev[1]
  src_rat = edge_index_rates[0]
  dst_rat = edge_index_rates[1]

  cnt_u = _count(dst_rev)
  cnt_m = _count(dst_rat)

  sum1 = _segsum(x_movie, src_rev, dst_rev)
  sum2 = _segsum(x_user, src_rat, dst_rat)

  user_x = _dense(sum1, cnt_u, x_user, W1l, b1l, W1r)
  movie_x, out_movie = _dense(sum2, cnt_m, x_movie, W2l, b2l, W2r,
                              Wlin2, blin2)

  sum3 = _segsum(movie_x, src_rev, dst_rev)
  _, out_user = _dense(sum3, cnt_u, user_x, W3l, b3l, W3r, Wlin1, blin1)

  return (out_user, out_movie)
